# Initial kernel scaffold; baseline (speedup 1.0000x reference)
import jax
import jax.numpy as jnp
from jax.experimental import pallas as pl


def kernel(params, x, x_clique, edge_index_graph, edge_attr_graph, atom2clique_index, batch):
    raise NotImplementedError("write your pallas kernel here")


# speedup vs baseline: 5.6709x; 5.6709x over previous
import jax
import jax.numpy as jnp
from jax import lax
from jax.experimental import pallas as pl
from jax.experimental.pallas import tpu as pltpu
from jax.experimental.pallas import tpu_sc as plsc

_N = 50000
_E = 800000
_C = 20000
_H = 64
_L = 3
_B = 128

_NS = 16
_K = 400
_CH = _NS * _K
_BLK = 512
_NP = 50176
_CP = 20480


def _cm(n, m):
    return ((n + m - 1) // m) * m


def _sc_segsum(n_tables, do_relu, ep, r2, name):
    nchunk = ep // _CH
    nz = r2 // _CH

    def body(*args):
        tabs = args[: 2 * n_tables]
        idxs = args[2 * n_tables: 3 * n_tables]
        dst_h = args[3 * n_tables]
        oa = args[3 * n_tables + 1]
        ob = args[3 * n_tables + 2]
        scr = args[3 * n_tables + 3:]
        idxv = scr[:n_tables]
        dstv = scr[n_tables]
        rows = scr[n_tables + 1: 2 * n_tables + 1]
        acc = scr[2 * n_tables + 1]
        sem = scr[2 * n_tables + 2]

        c = lax.axis_index("c")
        s = lax.axis_index("s")

        def zb(k, carry):
            rows[0][k, pl.ds(0, 16)] = jnp.zeros((16,), jnp.float32)
            rows[0][k, pl.ds(16, 16)] = jnp.zeros((16,), jnp.float32)
            return carry

        lax.fori_loop(0, _K, zb, 0)

        def zacc(z, carry):
            off = pl.multiple_of((s * nz + z) * _K, 8)
            pltpu.sync_copy(rows[0], acc.at[pl.ds(off, _K)])
            return carry

        lax.fori_loop(0, nz, zacc, 0)
        plsc.subcore_barrier()

        def main(j, carry):
            off = pl.multiple_of((s * nchunk + j) * _K, 8)
            for t in range(n_tables):
                pltpu.sync_copy(idxs[t].at[pl.ds(off, _K)], idxv[t])
            pltpu.sync_copy(dst_h.at[pl.ds(off, _K)], dstv)
            for t in range(n_tables):
                @pl.when(c == 0)
                def _g0(t=t):
                    pltpu.async_copy(tabs[2 * t].at[idxv[t]], rows[t], sem).wait()

                @pl.when(c == 1)
                def _g1(t=t):
                    pltpu.async_copy(tabs[2 * t + 1].at[idxv[t]], rows[t], sem).wait()
            if n_tables == 2:
                def cb(k, cc):
                    for hh in (0, 16):
                        v = rows[0][k, pl.ds(hh, 16)] + rows[1][k, pl.ds(hh, 16)]
                        if do_relu:
                            v = jnp.maximum(v, 0.0)
                        rows[0][k, pl.ds(hh, 16)] = v
                    return cc

                lax.fori_loop(0, _K, cb, 0)
            pltpu.sync_copy(rows[0], acc.at[dstv], add=True)
            return carry

        lax.fori_loop(0, nchunk, main, 0)
        plsc.subcore_barrier()

        def cout(z, carry):
            off = pl.multiple_of((s * nz + z) * _K, 8)
            pltpu.sync_copy(acc.at[pl.ds(off, _K)], rows[0])

            @pl.when(c == 0)
            def _o0():
                pltpu.sync_copy(rows[0], oa.at[pl.ds(off, _K)])

            @pl.when(c == 1)
            def _o1():
                pltpu.sync_copy(rows[0], ob.at[pl.ds(off, _K)])
            return carry

        lax.fori_loop(0, nz, cout, 0)

    scratch = (
        [pltpu.VMEM((_K,), jnp.int32) for _ in range(n_tables)]
        + [pltpu.VMEM((_K,), jnp.int32)]
        + [pltpu.VMEM((_K, 32), jnp.float32) for _ in range(n_tables)]
        + [pltpu.VMEM_SHARED((r2, 32), jnp.float32), pltpu.SemaphoreType.DMA]
    )
    return pl.kernel(
        body,
        out_type=(
            jax.ShapeDtypeStruct((r2, 32), jnp.float32),
            jax.ShapeDtypeStruct((r2, 32), jnp.float32),
        ),
        mesh=plsc.VectorSubcoreMesh(core_axis_name="c", subcore_axis_name="s",
                                    num_cores=2, num_subcores=_NS),
        scratch_types=scratch,
        compiler_params=pltpu.CompilerParams(use_tc_tiling_on_sc=False),
        name=name,
    )


def _row_mask(i, nvalid):
    rows = i * _BLK + lax.broadcasted_iota(jnp.int32, (_BLK, 1), 0)
    return (rows < nvalid).astype(jnp.float32)


def _acc_stats(st_ref, i, blk):
    @pl.when(i == 0)
    def _():
        st_ref[...] = blk

    @pl.when(i != 0)
    def _():
        st_ref[...] = st_ref[...] + blk


def _k_gine_pre(hs, ha, hb, aa, ab, w, b, u_ref, st_ref):
    i = pl.program_id(0)
    h = jnp.concatenate([ha[...], hb[...]], axis=1)
    a = h * hs[0, 0] + jnp.concatenate([aa[...], ab[...]], axis=1)
    u = jnp.dot(a, w[...], preferred_element_type=jnp.float32) + b[...]
    u_ref[...] = u
    m = _row_mask(i, _N)
    um = u * m
    s0 = jnp.sum(um, axis=0, keepdims=True)
    s1 = jnp.sum(um * u, axis=0, keepdims=True)
    blk = jnp.concatenate([s0, s1, jnp.zeros((6, 128), jnp.float32)], axis=0)
    _acc_stats(st_ref, i, blk)


def _k_gine_mid(u, sc1, sh1, w, b, v_ref, st_ref):
    i = pl.program_id(0)
    r = jnp.maximum(u[...] * sc1[...] + sh1[...], 0.0)
    v = jnp.dot(r, w[...], preferred_element_type=jnp.float32) + b[...]
    v_ref[...] = v
    m = _row_mask(i, _N)
    vm = v * m
    s0 = jnp.sum(vm, axis=0, keepdims=True)
    s1 = jnp.sum(vm * v, axis=0, keepdims=True)
    row = jnp.concatenate([s0, s1], axis=1)
    blk = jnp.concatenate([row, jnp.zeros((7, 128), jnp.float32)], axis=0)
    _acc_stats(st_ref, i, blk)


def _k_affine_relu_split(v, sc, sh, oa_ref, ob_ref):
    h = jnp.maximum(v[...] * sc[...] + sh[...], 0.0)
    oa_ref[...] = h[:, :32]
    ob_ref[...] = h[:, 32:]


def _k_clique_pre(wg, aa, ab, xa, xb, w, b, xn_ref, st_ref):
    i = pl.program_id(0)
    agg = jnp.concatenate([aa[...], ab[...]], axis=1)
    x0 = jnp.concatenate([xa[...], xb[...]], axis=1)
    t = jnp.maximum(jnp.dot(agg, w[...], preferred_element_type=jnp.float32) + b[...], 0.0)
    xn = x0 + wg[0, 0] * t
    xn_ref[...] = xn
    m = _row_mask(i, _C)
    xm = xn * m
    s0 = jnp.sum(xm, axis=0, keepdims=True)
    s1 = jnp.sum(xm * xn, axis=0, keepdims=True)
    row = jnp.concatenate([s0, s1], axis=1)
    blk = jnp.concatenate([row, jnp.zeros((7, 128), jnp.float32)], axis=0)
    _acc_stats(st_ref, i, blk)


def _k_affine_split(xn, sc, sh, oa_ref, ob_ref):
    t = xn[...] * sc[...] + sh[...]
    oa_ref[...] = t[:, :32]
    ob_ref[...] = t[:, 32:]


def _k_c2a_post(wt, ha, hb, ca, cb, w, b, oa_ref, ob_ref):
    agg = jnp.concatenate([ca[...], cb[...]], axis=1)
    t = jnp.dot(agg, w[...], preferred_element_type=jnp.float32) + b[...]
    t = jnp.where(t > 0.0, t, 0.01 * t)
    hn = jnp.concatenate([ha[...], hb[...]], axis=1) + wt[0, 0] * t
    oa_ref[...] = hn[:, :32]
    ob_ref[...] = hn[:, 32:]


def _vspec(shape):
    return pl.BlockSpec(shape, lambda i: (i, 0))


def _cspec(shape):
    return pl.BlockSpec(shape, lambda i: (0, 0))


_SMEM = pl.BlockSpec(memory_space=pltpu.SMEM)


def _moments_affine(st_row, n, g, bb):
    s0, s1 = st_row
    mean = s0 / n
    var = s1 / n - mean * mean
    sc = g * lax.rsqrt(var + 1e-5)
    sh = bb - mean * sc
    return sc[None, :], sh[None, :]


def _pad1(a, n, val):
    return jnp.concatenate([a, jnp.full((n - a.shape[0],), val, a.dtype)])


def kernel(params, x, x_clique, edge_index_graph, edge_attr_graph, atom2clique_index, batch):
    p = params
    f32 = jnp.float32
    i32 = jnp.int32

    atab = p['atom_emb'].reshape(9 * 64, _H)
    ctab0 = p['clique_emb'] @ p['W_clique'] + p['b_clique']
    btabs = []
    for i in range(_L):
        b0, b1, b2 = p['bond_emb'][i, 0], p['bond_emb'][i, 1], p['bond_emb'][i, 2]
        btabs.append((b0[:, None, None, :] + b1[None, :, None, :]
                      + b2[None, None, :, :]).reshape(512, _H))

    def halves(t):
        return jnp.asarray(t[:, :32], f32), jnp.asarray(t[:, 32:], f32)

    atab_a, atab_b = halves(atab)
    ctab0_a, ctab0_b = halves(ctab0)
    btab_h = [halves(t) for t in btabs]

    x = x.astype(i32)
    ea = edge_attr_graph.astype(i32)
    src = edge_index_graph[0].astype(i32)
    dst = edge_index_graph[1].astype(i32)
    row = atom2clique_index[0].astype(i32)
    col = atom2clique_index[1].astype(i32)
    batch = batch.astype(i32)
    cidx = ea[:, 0] * 64 + ea[:, 1] * 8 + ea[:, 2]

    ep_at = _cm(9 * _N + 1, _CH)
    at_idx = _pad1((x + 64 * jnp.arange(9, dtype=i32)[None, :]).T.reshape(-1), ep_at, 0)
    at_dst = _pad1(jnp.tile(jnp.arange(_N, dtype=i32), 9), ep_at, _N)

    ep_n = _cm(_N + 1, _CH)
    ep_c = _cm(_C + 1, _CH)
    r2_n = _cm(_N + 1, _CH)
    r2_c = _cm(_C + 1, _CH)
    r2_b = _CH

    cl_idx = _pad1(x_clique.astype(i32), ep_c, 0)
    cl_dst = _pad1(jnp.arange(_C, dtype=i32), ep_c, _C)
    a2c_row = _pad1(row, ep_n, 0)
    a2c_col = _pad1(col, ep_n, _C)
    c2a_col = _pad1(col, ep_n, 0)
    c2a_row = _pad1(row, ep_n, _N)
    ro_idx = _pad1(jnp.arange(_N, dtype=i32), ep_n, 0)
    ro_dst = _pad1(batch, ep_n, _B)

    sc_atom = _sc_segsum(1, False, ep_at, r2_n, "sc_atom_enc")
    sc_clq = _sc_segsum(1, False, ep_c, r2_c, "sc_clique_enc")
    sc_edge = _sc_segsum(2, True, _E, r2_n, "sc_edge_agg")
    sc_a2c = _sc_segsum(1, False, ep_n, r2_c, "sc_a2c")
    sc_c2a = _sc_segsum(1, False, ep_n, r2_n, "sc_c2a")
    sc_ro = _sc_segsum(1, False, ep_n, r2_b, "sc_readout")

    h0a, h0b = sc_atom(atab_a, atab_b, at_idx, at_dst)
    xca, xcb = sc_clq(ctab0_a, ctab0_b, cl_idx, cl_dst)

    gn = 98
    gc = 40

    ha, hb = h0a, h0b
    for i in range(_L):
        bta, btb = btab_h[i]
        aga, agb = sc_edge(ha, hb, bta, btb, src, cidx, dst)

        hs = (1.0 + p['eps'][i]).reshape(1, 1).astype(f32)
        u, st1 = pl.pallas_call(
            _k_gine_pre,
            grid=(gn,),
            in_specs=[_SMEM, _vspec((_BLK, 32)), _vspec((_BLK, 32)),
                      _vspec((_BLK, 32)), _vspec((_BLK, 32)),
                      _cspec((64, 128)), _cspec((1, 128))],
            out_specs=[_vspec((_BLK, 128)), _cspec((8, 128))],
            out_shape=[jax.ShapeDtypeStruct((_NP, 128), f32),
                       jax.ShapeDtypeStruct((8, 128), f32)],
        )(hs, ha, hb, aga, agb, p['W1'][i], p['b1'][i][None, :])
        sc1, sh1 = _moments_affine((st1[0], st1[1]), _N, p['bn1_g'][i], p['bn1_b'][i])

        v, st2 = pl.pallas_call(
            _k_gine_mid,
            grid=(gn,),
            in_specs=[_vspec((_BLK, 128)), _cspec((1, 128)), _cspec((1, 128)),
                      _cspec((128, 64)), _cspec((1, 64))],
            out_specs=[_vspec((_BLK, 64)), _cspec((8, 128))],
            out_shape=[jax.ShapeDtypeStruct((_NP, 64), f32),
                       jax.ShapeDtypeStruct((8, 128), f32)],
        )(u, sc1, sh1, p['W2'][i], p['b2'][i][None, :])
        sc2, sh2 = _moments_affine((st2[0, :64], st2[0, 64:]), _N,
                                   p['gn_g'][i], p['gn_b'][i])

        ha, hb = pl.pallas_call(
            _k_affine_relu_split,
            grid=(gn,),
            in_specs=[_vspec((_BLK, 64)), _cspec((1, 64)), _cspec((1, 64))],
            out_specs=[_vspec((_BLK, 32)), _vspec((_BLK, 32))],
            out_shape=[jax.ShapeDtypeStruct((_NP, 32), f32),
                       jax.ShapeDtypeStruct((_NP, 32), f32)],
        )(v, sc2, sh2)

        ca, cb = sc_a2c(ha, hb, a2c_row, a2c_col)
        wg = p['w_g2t'].reshape(1, 1).astype(f32)
        xn, st3 = pl.pallas_call(
            _k_clique_pre,
            grid=(gc,),
            in_specs=[_SMEM, _vspec((_BLK, 32)), _vspec((_BLK, 32)),
                      _vspec((_BLK, 32)), _vspec((_BLK, 32)),
                      _cspec((64, 64)), _cspec((1, 64))],
            out_specs=[_vspec((_BLK, 64)), _cspec((8, 128))],
            out_shape=[jax.ShapeDtypeStruct((_CP, 64), f32),
                       jax.ShapeDtypeStruct((8, 128), f32)],
        )(wg, ca, cb, xca, xcb, p['Wa2c'][i], p['ba2c'][i][None, :])
        sc3, sh3 = _moments_affine((st3[0, :64], st3[0, 64:]), _C,
                                   p['sn_g'][i], p['sn_b'][i])

        xca, xcb = pl.pallas_call(
            _k_affine_split,
            grid=(gc,),
            in_specs=[_vspec((_BLK, 64)), _cspec((1, 64)), _cspec((1, 64))],
            out_specs=[_vspec((_BLK, 32)), _vspec((_BLK, 32))],
            out_shape=[jax.ShapeDtypeStruct((_CP, 32), f32),
                       jax.ShapeDtypeStruct((_CP, 32), f32)],
        )(xn, sc3, sh3)

        ga, gb = sc_c2a(xca, xcb, c2a_col, c2a_row)
        wt = p['w_t2g'].reshape(1, 1).astype(f32)
        ha, hb = pl.pallas_call(
            _k_c2a_post,
            grid=(gn,),
            in_specs=[_SMEM, _vspec((_BLK, 32)), _vspec((_BLK, 32)),
                      _vspec((_BLK, 32)), _vspec((_BLK, 32)),
                      _cspec((64, 64)), _cspec((1, 64))],
            out_specs=[_vspec((_BLK, 32)), _vspec((_BLK, 32))],
            out_shape=[jax.ShapeDtypeStruct((_NP, 32), f32),
                       jax.ShapeDtypeStruct((_NP, 32), f32)],
        )(wt, ha, hb, ga, gb, p['Wc2a'][i], p['bc2a'][i][None, :])

    ra, rb = sc_ro(ha, hb, ro_idx, ro_dst)

    xc = jnp.concatenate([xca[:_C], xcb[:_C]], axis=1)
    graph_emb = jnp.concatenate([h0a[:_N], h0b[:_N]], axis=1)
    readout = jnp.concatenate([ra[:_B], rb[:_B]], axis=1)
    return (xc, graph_emb, readout)


# speedup vs baseline: 6.5940x; 1.1628x over previous
import jax
import jax.numpy as jnp
from jax import lax
from jax.experimental import pallas as pl
from jax.experimental.pallas import tpu as pltpu
from jax.experimental.pallas import tpu_sc as plsc

_N = 50000
_E = 800000
_C = 20000
_H = 64
_L = 3
_B = 128

_NS = 16
_K = 400
_CH = _NS * _K
_BLK = 512
_NP = 50176
_CP = 20480


def _cm(n, m):
    return ((n + m - 1) // m) * m


def _zero_fill(acc, buf, s, nz, kz):
    def zb(k, carry):
        buf[k, pl.ds(0, 16)] = jnp.zeros((16,), jnp.float32)
        buf[k, pl.ds(16, 16)] = jnp.zeros((16,), jnp.float32)
        return carry

    lax.fori_loop(0, kz, zb, 0)

    def zacc(z, carry):
        off = pl.multiple_of((s * nz + z) * kz, 8)
        pltpu.sync_copy(buf.at[pl.ds(0, kz)], acc.at[pl.ds(off, kz)])
        return carry

    lax.fori_loop(0, nz, zacc, 0)


def _copy_out(acc, buf, s, nz, kz, c, oa, ob):
    def cout(z, carry):
        off = pl.multiple_of((s * nz + z) * kz, 8)
        pltpu.sync_copy(acc.at[pl.ds(off, kz)], buf.at[pl.ds(0, kz)])

        @pl.when(c == 0)
        def _o0():
            pltpu.sync_copy(buf.at[pl.ds(0, kz)], oa.at[pl.ds(off, kz)])

        @pl.when(c == 1)
        def _o1():
            pltpu.sync_copy(buf.at[pl.ds(0, kz)], ob.at[pl.ds(off, kz)])
        return carry

    lax.fori_loop(0, nz, cout, 0)


def _sc_segsum(n_tables, do_relu, ep, r2, kc, name):
    assert n_tables == 1
    nchunk = ep // (_NS * kc)
    kz = min(kc, r2 // _NS)
    nz = r2 // (_NS * kz)
    assert r2 % (_NS * kz) == 0 and kz % 8 == 0

    def body(tab_a, tab_b, idx_h, dst_h, oa, ob, idxv, dstv, rows, acc, sem):
        c = lax.axis_index("c")
        s = lax.axis_index("s")
        _zero_fill(acc, rows, s, nz, kz)
        plsc.subcore_barrier()

        def main(j, carry):
            off = pl.multiple_of((s * nchunk + j) * kc, 8)
            pltpu.sync_copy(idx_h.at[pl.ds(off, kc)], idxv)
            pltpu.sync_copy(dst_h.at[pl.ds(off, kc)], dstv)

            @pl.when(c == 0)
            def _g0():
                pltpu.async_copy(tab_a.at[idxv], rows, sem).wait()

            @pl.when(c == 1)
            def _g1():
                pltpu.async_copy(tab_b.at[idxv], rows, sem).wait()
            pltpu.sync_copy(rows, acc.at[dstv], add=True)
            return carry

        lax.fori_loop(0, nchunk, main, 0)
        plsc.subcore_barrier()
        _copy_out(acc, rows, s, nz, kz, c, oa, ob)

    scratch = [
        pltpu.VMEM((kc,), jnp.int32),
        pltpu.VMEM((kc,), jnp.int32),
        pltpu.VMEM((kc, 32), jnp.float32),
        pltpu.VMEM_SHARED((r2, 32), jnp.float32),
        pltpu.SemaphoreType.DMA,
    ]
    return pl.kernel(
        body,
        out_type=(
            jax.ShapeDtypeStruct((r2, 32), jnp.float32),
            jax.ShapeDtypeStruct((r2, 32), jnp.float32),
        ),
        mesh=plsc.VectorSubcoreMesh(core_axis_name="c", subcore_axis_name="s",
                                    num_cores=2, num_subcores=_NS),
        scratch_types=scratch,
        compiler_params=pltpu.CompilerParams(use_tc_tiling_on_sc=False),
        name=name,
    )


def _sc_edge(ep, r2, name):
    kc = 200
    nchunk = ep // (_NS * kc)
    assert nchunk % 2 == 0 and nchunk >= 4
    kz = min(kc, r2 // _NS)
    nz = r2 // (_NS * kz)
    assert r2 % (_NS * kz) == 0

    def body(t0a, t0b, t1a, t1b, idx0_h, idx1_h, dst_h, oa, ob,
             i0v0, i0v1, i1v0, i1v1, dstv, r0b0, r0b1, r1b0, r1b1,
             acc, sem0, sem1):
        c = lax.axis_index("c")
        s = lax.axis_index("s")
        i0 = (i0v0, i0v1)
        i1 = (i1v0, i1v1)
        r0 = (r0b0, r0b1)
        r1 = (r1b0, r1b1)
        sems = (sem0, sem1)
        _zero_fill(acc, r0b0, s, nz, kz)
        plsc.subcore_barrier()

        def chunk_off(j):
            return pl.multiple_of((s * nchunk + j) * kc, 8)

        def load_gidx(j, b):
            off = chunk_off(j)
            pltpu.sync_copy(idx0_h.at[pl.ds(off, kc)], i0[b])
            pltpu.sync_copy(idx1_h.at[pl.ds(off, kc)], i1[b])

        def start_g(b):
            @pl.when(c == 0)
            def _g0():
                pltpu.async_copy(t0a.at[i0[b]], r0[b], sems[b])
                pltpu.async_copy(t1a.at[i1[b]], r1[b], sems[b])

            @pl.when(c == 1)
            def _g1():
                pltpu.async_copy(t0b.at[i0[b]], r0[b], sems[b])
                pltpu.async_copy(t1b.at[i1[b]], r1[b], sems[b])

        def finish(j, b):
            pltpu.sync_copy(dst_h.at[pl.ds(chunk_off(j), kc)], dstv)
            pltpu.make_async_copy(t0a.at[i0[b]], r0[b], sems[b]).wait()
            pltpu.make_async_copy(t1a.at[i1[b]], r1[b], sems[b]).wait()
            ra, rb = r0[b], r1[b]

            def cb(k8, cc):
                for u in range(8):
                    for hh in (0, 16):
                        k = k8 * 8 + u
                        v = ra[k, pl.ds(hh, 16)] + rb[k, pl.ds(hh, 16)]
                        ra[k, pl.ds(hh, 16)] = jnp.maximum(v, 0.0)
                return cc

            lax.fori_loop(0, kc // 8, cb, 0)
            pltpu.sync_copy(ra, acc.at[dstv], add=True)

        load_gidx(0, 0)
        start_g(0)

        def pair(j2, carry):
            j = 2 * j2 + 1
            load_gidx(j, 1)
            start_g(1)
            finish(j - 1, 0)
            load_gidx(j + 1, 0)
            start_g(0)
            finish(j, 1)
            return carry

        lax.fori_loop(0, nchunk // 2 - 1, pair, 0)
        load_gidx(nchunk - 1, 1)
        start_g(1)
        finish(nchunk - 2, 0)
        finish(nchunk - 1, 1)
        plsc.subcore_barrier()
        _copy_out(acc, r0b0, s, nz, kz, c, oa, ob)

    scratch = (
        [pltpu.VMEM((kc,), jnp.int32) for _ in range(5)]
        + [pltpu.VMEM((kc, 32), jnp.float32) for _ in range(4)]
        + [pltpu.VMEM_SHARED((r2, 32), jnp.float32),
           pltpu.SemaphoreType.DMA, pltpu.SemaphoreType.DMA]
    )
    return pl.kernel(
        body,
        out_type=(
            jax.ShapeDtypeStruct((r2, 32), jnp.float32),
            jax.ShapeDtypeStruct((r2, 32), jnp.float32),
        ),
        mesh=plsc.VectorSubcoreMesh(core_axis_name="c", subcore_axis_name="s",
                                    num_cores=2, num_subcores=_NS),
        scratch_types=scratch,
        compiler_params=pltpu.CompilerParams(use_tc_tiling_on_sc=False),
        name=name,
    )


def _row_mask(i, nvalid):
    rows = i * _BLK + lax.broadcasted_iota(jnp.int32, (_BLK, 1), 0)
    return (rows < nvalid).astype(jnp.float32)


def _acc_stats(st_ref, i, blk):
    @pl.when(i == 0)
    def _():
        st_ref[...] = blk

    @pl.when(i != 0)
    def _():
        st_ref[...] = st_ref[...] + blk


def _k_gine_pre(hs, ha, hb, aa, ab, w, b, u_ref, st_ref):
    i = pl.program_id(0)
    h = jnp.concatenate([ha[...], hb[...]], axis=1)
    a = h * hs[0, 0] + jnp.concatenate([aa[...], ab[...]], axis=1)
    u = jnp.dot(a, w[...], preferred_element_type=jnp.float32) + b[...]
    u_ref[...] = u
    m = _row_mask(i, _N)
    um = u * m
    s0 = jnp.sum(um, axis=0, keepdims=True)
    s1 = jnp.sum(um * u, axis=0, keepdims=True)
    blk = jnp.concatenate([s0, s1, jnp.zeros((6, 128), jnp.float32)], axis=0)
    _acc_stats(st_ref, i, blk)


def _k_gine_mid(u, sc1, sh1, w, b, v_ref, st_ref):
    i = pl.program_id(0)
    r = jnp.maximum(u[...] * sc1[...] + sh1[...], 0.0)
    v = jnp.dot(r, w[...], preferred_element_type=jnp.float32) + b[...]
    v_ref[...] = v
    m = _row_mask(i, _N)
    vm = v * m
    s0 = jnp.sum(vm, axis=0, keepdims=True)
    s1 = jnp.sum(vm * v, axis=0, keepdims=True)
    row = jnp.concatenate([s0, s1], axis=1)
    blk = jnp.concatenate([row, jnp.zeros((7, 128), jnp.float32)], axis=0)
    _acc_stats(st_ref, i, blk)


def _k_affine_relu_split(v, sc, sh, oa_ref, ob_ref):
    h = jnp.maximum(v[...] * sc[...] + sh[...], 0.0)
    oa_ref[...] = h[:, :32]
    ob_ref[...] = h[:, 32:]


def _k_clique_pre(wg, aa, ab, xa, xb, w, b, xn_ref, st_ref):
    i = pl.program_id(0)
    agg = jnp.concatenate([aa[...], ab[...]], axis=1)
    x0 = jnp.concatenate([xa[...], xb[...]], axis=1)
    t = jnp.maximum(jnp.dot(agg, w[...], preferred_element_type=jnp.float32) + b[...], 0.0)
    xn = x0 + wg[0, 0] * t
    xn_ref[...] = xn
    m = _row_mask(i, _C)
    xm = xn * m
    s0 = jnp.sum(xm, axis=0, keepdims=True)
    s1 = jnp.sum(xm * xn, axis=0, keepdims=True)
    row = jnp.concatenate([s0, s1], axis=1)
    blk = jnp.concatenate([row, jnp.zeros((7, 128), jnp.float32)], axis=0)
    _acc_stats(st_ref, i, blk)


def _k_affine_split(xn, sc, sh, oa_ref, ob_ref):
    t = xn[...] * sc[...] + sh[...]
    oa_ref[...] = t[:, :32]
    ob_ref[...] = t[:, 32:]


def _k_c2a_post(wt, ha, hb, ca, cb, w, b, oa_ref, ob_ref):
    agg = jnp.concatenate([ca[...], cb[...]], axis=1)
    t = jnp.dot(agg, w[...], preferred_element_type=jnp.float32) + b[...]
    t = jnp.where(t > 0.0, t, 0.01 * t)
    hn = jnp.concatenate([ha[...], hb[...]], axis=1) + wt[0, 0] * t
    oa_ref[...] = hn[:, :32]
    ob_ref[...] = hn[:, 32:]


def _vspec(shape):
    return pl.BlockSpec(shape, lambda i: (i, 0))


def _cspec(shape):
    return pl.BlockSpec(shape, lambda i: (0, 0))


_SMEM = pl.BlockSpec(memory_space=pltpu.SMEM)


def _moments_affine(st_row, n, g, bb):
    s0, s1 = st_row
    mean = s0 / n
    var = s1 / n - mean * mean
    sc = g * lax.rsqrt(var + 1e-5)
    sh = bb - mean * sc
    return sc[None, :], sh[None, :]


def _pad1(a, n, val):
    return jnp.concatenate([a, jnp.full((n - a.shape[0],), val, a.dtype)])


def kernel(params, x, x_clique, edge_index_graph, edge_attr_graph, atom2clique_index, batch):
    p = params
    f32 = jnp.float32
    i32 = jnp.int32

    atab = p['atom_emb'].reshape(9 * 64, _H)
    ctab0 = p['clique_emb'] @ p['W_clique'] + p['b_clique']
    btabs = []
    for i in range(_L):
        b0, b1, b2 = p['bond_emb'][i, 0], p['bond_emb'][i, 1], p['bond_emb'][i, 2]
        btabs.append((b0[:, None, None, :] + b1[None, :, None, :]
                      + b2[None, None, :, :]).reshape(512, _H))

    def halves(t):
        return jnp.asarray(t[:, :32], f32), jnp.asarray(t[:, 32:], f32)

    atab_a, atab_b = halves(atab)
    ctab0_a, ctab0_b = halves(ctab0)
    btab_h = [halves(t) for t in btabs]

    x = x.astype(i32)
    ea = edge_attr_graph.astype(i32)
    src = edge_index_graph[0].astype(i32)
    dst = edge_index_graph[1].astype(i32)
    row = atom2clique_index[0].astype(i32)
    col = atom2clique_index[1].astype(i32)
    batch = batch.astype(i32)
    cidx = ea[:, 0] * 64 + ea[:, 1] * 8 + ea[:, 2]

    ep_at = _cm(9 * _N + 1, _NS * 800)
    at_idx = _pad1((x + 64 * jnp.arange(9, dtype=i32)[None, :]).T.reshape(-1), ep_at, 0)
    at_dst = _pad1(jnp.tile(jnp.arange(_N, dtype=i32), 9), ep_at, _N)

    ep_n = _cm(_N + 1, _CH)
    ep_c = _cm(_C + 1, _CH)
    r2_n = _cm(_N + 1, _CH)
    r2_c = _cm(_C + 1, _CH)
    r2_b = _CH

    cl_idx = _pad1(x_clique.astype(i32), ep_c, 0)
    cl_dst = _pad1(jnp.arange(_C, dtype=i32), ep_c, _C)
    a2c_row = _pad1(row, ep_n, 0)
    a2c_col = _pad1(col, ep_n, _C)
    c2a_col = _pad1(col, ep_n, 0)
    c2a_row = _pad1(row, ep_n, _N)
    ro_idx = _pad1(jnp.arange(_N, dtype=i32), ep_n, 0)
    ro_dst = _pad1(batch, ep_n, _B)

    sc_atom = _sc_segsum(1, False, ep_at, r2_n, 800, "sc_atom_enc")
    sc_clq = _sc_segsum(1, False, ep_c, r2_c, 1600, "sc_clique_enc")
    sc_edge = _sc_edge(_E, r2_n, "sc_edge_agg")
    sc_a2c = _sc_segsum(1, False, ep_n, r2_c, 1600, "sc_a2c")
    sc_c2a = _sc_segsum(1, False, ep_n, r2_n, 800, "sc_c2a")
    sc_ro = _sc_segsum(1, False, ep_n, r2_b, 3200, "sc_readout")

    h0a, h0b = sc_atom(atab_a, atab_b, at_idx, at_dst)
    xca, xcb = sc_clq(ctab0_a, ctab0_b, cl_idx, cl_dst)

    gn = 98
    gc = 40

    ha, hb = h0a, h0b
    for i in range(_L):
        bta, btb = btab_h[i]
        aga, agb = sc_edge(ha, hb, bta, btb, src, cidx, dst)

        hs = (1.0 + p['eps'][i]).reshape(1, 1).astype(f32)
        u, st1 = pl.pallas_call(
            _k_gine_pre,
            grid=(gn,),
            in_specs=[_SMEM, _vspec((_BLK, 32)), _vspec((_BLK, 32)),
                      _vspec((_BLK, 32)), _vspec((_BLK, 32)),
                      _cspec((64, 128)), _cspec((1, 128))],
            out_specs=[_vspec((_BLK, 128)), _cspec((8, 128))],
            out_shape=[jax.ShapeDtypeStruct((_NP, 128), f32),
                       jax.ShapeDtypeStruct((8, 128), f32)],
        )(hs, ha, hb, aga, agb, p['W1'][i], p['b1'][i][None, :])
        sc1, sh1 = _moments_affine((st1[0], st1[1]), _N, p['bn1_g'][i], p['bn1_b'][i])

        v, st2 = pl.pallas_call(
            _k_gine_mid,
            grid=(gn,),
            in_specs=[_vspec((_BLK, 128)), _cspec((1, 128)), _cspec((1, 128)),
                      _cspec((128, 64)), _cspec((1, 64))],
            out_specs=[_vspec((_BLK, 64)), _cspec((8, 128))],
            out_shape=[jax.ShapeDtypeStruct((_NP, 64), f32),
                       jax.ShapeDtypeStruct((8, 128), f32)],
        )(u, sc1, sh1, p['W2'][i], p['b2'][i][None, :])
        sc2, sh2 = _moments_affine((st2[0, :64], st2[0, 64:]), _N,
                                   p['gn_g'][i], p['gn_b'][i])

        ha, hb = pl.pallas_call(
            _k_affine_relu_split,
            grid=(gn,),
            in_specs=[_vspec((_BLK, 64)), _cspec((1, 64)), _cspec((1, 64))],
            out_specs=[_vspec((_BLK, 32)), _vspec((_BLK, 32))],
            out_shape=[jax.ShapeDtypeStruct((_NP, 32), f32),
                       jax.ShapeDtypeStruct((_NP, 32), f32)],
        )(v, sc2, sh2)

        ca, cb = sc_a2c(ha, hb, a2c_row, a2c_col)
        wg = p['w_g2t'].reshape(1, 1).astype(f32)
        xn, st3 = pl.pallas_call(
            _k_clique_pre,
            grid=(gc,),
            in_specs=[_SMEM, _vspec((_BLK, 32)), _vspec((_BLK, 32)),
                      _vspec((_BLK, 32)), _vspec((_BLK, 32)),
                      _cspec((64, 64)), _cspec((1, 64))],
            out_specs=[_vspec((_BLK, 64)), _cspec((8, 128))],
            out_shape=[jax.ShapeDtypeStruct((_CP, 64), f32),
                       jax.ShapeDtypeStruct((8, 128), f32)],
        )(wg, ca, cb, xca, xcb, p['Wa2c'][i], p['ba2c'][i][None, :])
        sc3, sh3 = _moments_affine((st3[0, :64], st3[0, 64:]), _C,
                                   p['sn_g'][i], p['sn_b'][i])

        xca, xcb = pl.pallas_call(
            _k_affine_split,
            grid=(gc,),
            in_specs=[_vspec((_BLK, 64)), _cspec((1, 64)), _cspec((1, 64))],
            out_specs=[_vspec((_BLK, 32)), _vspec((_BLK, 32))],
            out_shape=[jax.ShapeDtypeStruct((_CP, 32), f32),
                       jax.ShapeDtypeStruct((_CP, 32), f32)],
        )(xn, sc3, sh3)

        ga, gb = sc_c2a(xca, xcb, c2a_col, c2a_row)
        wt = p['w_t2g'].reshape(1, 1).astype(f32)
        ha, hb = pl.pallas_call(
            _k_c2a_post,
            grid=(gn,),
            in_specs=[_SMEM, _vspec((_BLK, 32)), _vspec((_BLK, 32)),
                      _vspec((_BLK, 32)), _vspec((_BLK, 32)),
                      _cspec((64, 64)), _cspec((1, 64))],
            out_specs=[_vspec((_BLK, 32)), _vspec((_BLK, 32))],
            out_shape=[jax.ShapeDtypeStruct((_NP, 32), f32),
                       jax.ShapeDtypeStruct((_NP, 32), f32)],
        )(wt, ha, hb, ga, gb, p['Wc2a'][i], p['bc2a'][i][None, :])

    ra, rb = sc_ro(ha, hb, ro_idx, ro_dst)

    xc = jnp.concatenate([xca[:_C], xcb[:_C]], axis=1)
    graph_emb = jnp.concatenate([h0a[:_N], h0b[:_N]], axis=1)
    readout = jnp.concatenate([ra[:_B], rb[:_B]], axis=1)
    return (xc, graph_emb, readout)
